# baseline (speedup 1.0000x reference)
import functools

import jax
import jax.numpy as jnp
from jax import lax
from jax.experimental import pallas as pl
from jax.experimental.pallas import tpu as pltpu
from jax.experimental.pallas import tpu_sc as plsc

NN = 10000
DD = 128
EE = 320000
CH = 128
NCH = EE // CH
NC, NS = 2, 16
NW = NC * NS
MAXI = (NCH + NW - 1) // NW
ACC_ROWS = 10112
RPT = ACC_ROWS // NS
DEG_ROWS = 10240
DPT = DEG_ROWS // NS
ZR = 64
DUMMY = NN

_mesh = plsc.VectorSubcoreMesh(
    core_axis_name="c", subcore_axis_name="s", num_cores=NC, num_subcores=NS
)


def _norm_body(x_ref, o_ref):
    v = x_ref[...]
    n = jnp.sqrt(jnp.sum(v * v, axis=1, keepdims=True))
    o_ref[...] = v / jnp.maximum(n, 1e-12)


def _scale_body(x_ref, dd_ref, y_out):
    dis = lax.rsqrt(dd_ref[0] + dd_ref[1])
    y_out[...] = dis * x_ref[...]


def _combine_mid_body(acc_ref, dd_ref, x_ref, t_ref, y_ref):
    dis = lax.rsqrt(dd_ref[0] + dd_ref[1])
    h = dis * (acc_ref[0] + acc_ref[1])
    t_ref[...] = x_ref[...] + h
    y_ref[...] = dis * h


def _combine_fin_body(acc_ref, dd_ref, t_ref, o_ref):
    dis = lax.rsqrt(dd_ref[0] + dd_ref[1])
    o_ref[...] = t_ref[...] + dis * (acc_ref[0] + acc_ref[1])


_RB = 400


def _row_spec(width):
    return pl.BlockSpec((_RB, width), lambda i: (i, 0))


_dd_spec = pl.BlockSpec((NC, _RB, 1), lambda i: (0, i, 0))
_acc_spec = pl.BlockSpec((NC, _RB, DD), lambda i: (0, i, 0))
_f32 = functools.partial(jax.ShapeDtypeStruct, dtype=jnp.float32)


def _tc_norm(x):
    return pl.pallas_call(
        _norm_body,
        grid=(NN // _RB,),
        in_specs=[_row_spec(DD)],
        out_specs=_row_spec(DD),
        out_shape=_f32((NN, DD)),
    )(x)


def _tc_scale(x, dd):
    return pl.pallas_call(
        _scale_body,
        grid=(NN // _RB,),
        in_specs=[_row_spec(DD), _dd_spec],
        out_specs=_row_spec(DD),
        out_shape=_f32((NN, DD)),
    )(x, dd)


def _tc_combine_mid(acc, dd, x):
    return pl.pallas_call(
        _combine_mid_body,
        grid=(NN // _RB,),
        in_specs=[_acc_spec, _dd_spec, _row_spec(DD)],
        out_specs=(_row_spec(DD), _row_spec(DD)),
        out_shape=(_f32((NN, DD)), _f32((NN, DD))),
    )(acc, dd, x)


def _tc_combine_fin(acc, dd, t):
    return pl.pallas_call(
        _combine_fin_body,
        grid=(NN // _RB,),
        in_specs=[_acc_spec, _dd_spec, _row_spec(DD)],
        out_specs=_row_spec(DD),
        out_shape=_f32((NN, DD)),
    )(acc, dd, t)


def _deg_body(ei_hbm, deg_out,
              rb0, rb1, rb2, cb0, cb1, cb2, vb0, vb1, vb2, zbuf,
              deg_sh, is0, is1, is2):
    c = lax.axis_index("c")
    s = lax.axis_index("s")
    wid = s * NC + c
    rb = (rb0, rb1, rb2)
    cb = (cb0, cb1, cb2)
    vb = (vb0, vb1, vb2)
    isem = (is0, is1, is2)

    def _zz(i, _):
        zbuf[pl.ds(i * 16, 16)] = jnp.zeros((16,), jnp.float32)
        return 0

    lax.fori_loop(0, DPT // 16, _zz, 0)
    pltpu.sync_copy(zbuf, deg_sh.at[pl.ds(s * DPT, DPT)])
    plsc.subcore_barrier()

    def _issue_idx(k, b):
        off = k * CH
        pltpu.async_copy(ei_hbm.at[0, pl.ds(off, CH)], rb[b], isem[b])
        pltpu.async_copy(ei_hbm.at[1, pl.ds(off, CH)], cb[b], isem[b])

    _issue_idx(wid, 0)
    _issue_idx(wid + NW, 1)
    _issue_idx(wid + 2 * NW, 2)

    def _group(g, _):
        for b in range(3):
            k = wid + NW * (g * 3 + b)

            @pl.when(k < NCH)
            def _(b=b, k=k):
                pltpu.make_async_copy(ei_hbm.at[0, pl.ds(0, CH)], rb[b],
                                      isem[b]).wait()
                pltpu.make_async_copy(ei_hbm.at[1, pl.ds(0, CH)], cb[b],
                                      isem[b]).wait()

                def _vec(j, _):
                    r = rb[b][pl.ds(j * 16, 16)]
                    cc = cb[b][pl.ds(j * 16, 16)]
                    vb[b][pl.ds(j * 16, 16)] = jnp.where(r != cc, 1.0, 0.0)
                    return 0

                lax.fori_loop(0, CH // 16, _vec, 0)
                pltpu.sync_copy(vb[b], deg_sh.at[rb[b]], add=True)

                @pl.when(k + 3 * NW < NCH)
                def _():
                    _issue_idx(k + 3 * NW, b)

        return 0

    lax.fori_loop(0, _NG, _group, 0)
    plsc.subcore_barrier()
    pltpu.sync_copy(deg_sh.at[pl.ds(s * DPT, DPT)],
                    deg_out.at[c, pl.ds(s * DPT, DPT)])


_deg_kernel = functools.partial(
    pl.kernel,
    out_type=jax.ShapeDtypeStruct((NC, DEG_ROWS), jnp.float32),
    mesh=_mesh,
    scratch_types=(
        [pltpu.VMEM((CH,), jnp.int32)] * 6
        + [pltpu.VMEM((CH,), jnp.float32)] * 3
        + [pltpu.VMEM((DPT,), jnp.float32),
           pltpu.VMEM_SHARED((DEG_ROWS,), jnp.float32)]
        + [pltpu.SemaphoreType.DMA] * 3
    ),
)(_deg_body)


_NB = 3
_NG = (MAXI + _NB - 1) // _NB


def _spmm_body(y_hbm, ei_hbm, acc_out,
               rb0, rb1, rb2, cb0, cb1, cb2,
               mb0, mb1, mb2, acc_sh,
               is0, is1, is2, gs0, gs1, gs2, ss0, ss1, ss2):
    c = lax.axis_index("c")
    s = lax.axis_index("s")
    wid = s * NC + c
    rb = (rb0, rb1, rb2)
    cb = (cb0, cb1, cb2)
    mb = (mb0, mb1, mb2)
    isem = (is0, is1, is2)
    gsem = (gs0, gs1, gs2)
    ssem = (ss0, ss1, ss2)

    def _zz(i, _):
        def _zrow(j, _):
            mb0[i, pl.ds(j * 16, 16)] = jnp.zeros((16,), jnp.float32)
            return 0

        lax.fori_loop(0, DD // 16, _zrow, 0)
        return 0

    lax.fori_loop(0, CH, _zz, 0)

    def _zacc(k, _):
        pltpu.sync_copy(mb0, acc_sh.at[pl.ds(s * RPT + k * CH, CH)])
        return 0

    lax.fori_loop(0, RPT // CH, _zacc, 0)
    pltpu.sync_copy(mb0.at[pl.ds(0, RPT % CH)],
                    acc_sh.at[pl.ds(s * RPT + (RPT // CH) * CH, RPT % CH)])
    plsc.subcore_barrier()

    def _issue_idx(k, b):
        off = k * CH
        pltpu.async_copy(ei_hbm.at[0, pl.ds(off, CH)], rb[b], isem[b])
        pltpu.async_copy(ei_hbm.at[1, pl.ds(off, CH)], cb[b], isem[b])

    def _issue_gather(b):
        pltpu.make_async_copy(ei_hbm.at[0, pl.ds(0, CH)], rb[b],
                              isem[b]).wait()
        pltpu.make_async_copy(ei_hbm.at[1, pl.ds(0, CH)], cb[b],
                              isem[b]).wait()
        pltpu.async_copy(y_hbm.at[rb[b]], mb[b], gsem[b])

    _issue_idx(wid, 0)
    _issue_idx(wid + NW, 1)
    _issue_gather(0)
    _issue_idx(wid + 2 * NW, 2)
    _issue_gather(1)

    def _group(g, _):
        for b in range(_NB):
            i = g * _NB + b
            k = wid + NW * i

            @pl.when(k < NCH)
            def _(b=b, k=k):
                pltpu.make_async_copy(
                    y_hbm.at[pl.ds(0, CH)], mb[b], gsem[b]).wait()

                def _vec(j, _):
                    r = rb[b][pl.ds(j * 16, 16)]
                    cc = cb[b][pl.ds(j * 16, 16)]
                    cb[b][pl.ds(j * 16, 16)] = jnp.where(r != cc, cc, DUMMY)
                    return 0

                lax.fori_loop(0, CH // 16, _vec, 0)
                pltpu.sync_copy(mb[b], acc_sh.at[cb[b]], add=True)

                @pl.when(k + 3 * NW < NCH)
                def _():
                    _issue_idx(k + 3 * NW, b)

                @pl.when(k + 2 * NW < NCH)
                def _(b=b):
                    _issue_gather((b + 2) % _NB)

        return 0

    lax.fori_loop(0, _NG, _group, 0)
    plsc.subcore_barrier()
    pltpu.sync_copy(acc_sh.at[pl.ds(s * RPT, RPT)],
                    acc_out.at[c, pl.ds(s * RPT, RPT)])


_spmm_kernel = functools.partial(
    pl.kernel,
    out_type=jax.ShapeDtypeStruct((NC, ACC_ROWS, DD), jnp.float32),
    mesh=_mesh,
    scratch_types=(
        [pltpu.VMEM((CH,), jnp.int32)] * 6
        + [pltpu.VMEM((CH, DD), jnp.float32)] * 3
        + [pltpu.VMEM_SHARED((ACC_ROWS, DD), jnp.float32)]
        + [pltpu.SemaphoreType.DMA] * 9
    ),
)(_spmm_body)


@jax.jit
def _impl(edge_index, features, preference):
    ei = edge_index.astype(jnp.int32)
    xcat = jnp.concatenate([preference.astype(jnp.float32),
                            features.astype(jnp.float32)], axis=0)
    x = _tc_norm(xcat)
    deg_parts = _deg_kernel(ei)
    dd = deg_parts[:, :NN].reshape(NC, NN, 1)
    y1 = _tc_scale(x, dd)
    acc1 = _spmm_kernel(y1, ei)
    t1, y2 = _tc_combine_mid(acc1, dd, x)
    acc2 = _spmm_kernel(y2, ei)
    total = _tc_combine_fin(acc2, dd, t1)
    return total, preference


def kernel(edge_index, features, preference):
    return _impl(edge_index, features, preference)


# speedup vs baseline: 1.0824x; 1.0824x over previous
import functools

import jax
import jax.numpy as jnp
from jax import lax
from jax.experimental import pallas as pl
from jax.experimental.pallas import tpu as pltpu
from jax.experimental.pallas import tpu_sc as plsc

NN = 10000
DD = 128
EE = 320000
CH = 128
NCH = EE // CH
NC, NS = 2, 16
NW = NC * NS
MAXI = (NCH + NW - 1) // NW
ACC_ROWS = 10112
RPT = ACC_ROWS // NS
DEG_ROWS = 10240
DPT = DEG_ROWS // NS
ZR = 64
DUMMY = NN

_mesh = plsc.VectorSubcoreMesh(
    core_axis_name="c", subcore_axis_name="s", num_cores=NC, num_subcores=NS
)


def _norm_body(x_ref, o_ref):
    v = x_ref[...]
    n = jnp.sqrt(jnp.sum(v * v, axis=1, keepdims=True))
    o_ref[...] = v / jnp.maximum(n, 1e-12)


def _scale_body(x_ref, dd_ref, y_out):
    dis = lax.rsqrt(dd_ref[0] + dd_ref[1])
    y_out[...] = dis * x_ref[...]


def _combine_mid_body(acc_ref, dd_ref, x_ref, t_ref, y_ref):
    dis = lax.rsqrt(dd_ref[0] + dd_ref[1])
    h = dis * (acc_ref[0] + acc_ref[1])
    t_ref[...] = x_ref[...] + h
    y_ref[...] = dis * h


def _combine_fin_body(acc_ref, dd_ref, t_ref, o_ref):
    dis = lax.rsqrt(dd_ref[0] + dd_ref[1])
    o_ref[...] = t_ref[...] + dis * (acc_ref[0] + acc_ref[1])


_RB = 1000


def _row_spec(width):
    return pl.BlockSpec((_RB, width), lambda i: (i, 0))


_dd_spec = pl.BlockSpec((NC, _RB, 1), lambda i: (0, i, 0))
_acc_spec = pl.BlockSpec((NC, _RB, DD), lambda i: (0, i, 0))
_f32 = functools.partial(jax.ShapeDtypeStruct, dtype=jnp.float32)


def _tc_norm(x):
    return pl.pallas_call(
        _norm_body,
        grid=(NN // _RB,),
        in_specs=[_row_spec(DD)],
        out_specs=_row_spec(DD),
        out_shape=_f32((NN, DD)),
    )(x)


def _tc_scale(x, dd):
    return pl.pallas_call(
        _scale_body,
        grid=(NN // _RB,),
        in_specs=[_row_spec(DD), _dd_spec],
        out_specs=_row_spec(DD),
        out_shape=_f32((NN, DD)),
    )(x, dd)


def _tc_combine_mid(acc, dd, x):
    return pl.pallas_call(
        _combine_mid_body,
        grid=(NN // _RB,),
        in_specs=[_acc_spec, _dd_spec, _row_spec(DD)],
        out_specs=(_row_spec(DD), _row_spec(DD)),
        out_shape=(_f32((NN, DD)), _f32((NN, DD))),
    )(acc, dd, x)


def _tc_combine_fin(acc, dd, t):
    return pl.pallas_call(
        _combine_fin_body,
        grid=(NN // _RB,),
        in_specs=[_acc_spec, _dd_spec, _row_spec(DD)],
        out_specs=_row_spec(DD),
        out_shape=_f32((NN, DD)),
    )(acc, dd, t)


def _deg_body(ei_hbm, deg_out,
              rb0, rb1, rb2, cb0, cb1, cb2, vb0, vb1, vb2, zbuf,
              deg_sh, is0, is1, is2):
    c = lax.axis_index("c")
    s = lax.axis_index("s")
    wid = s * NC + c
    rb = (rb0, rb1, rb2)
    cb = (cb0, cb1, cb2)
    vb = (vb0, vb1, vb2)
    isem = (is0, is1, is2)

    def _zz(i, _):
        zbuf[pl.ds(i * 16, 16)] = jnp.zeros((16,), jnp.float32)
        return 0

    lax.fori_loop(0, DPT // 16, _zz, 0)
    pltpu.sync_copy(zbuf, deg_sh.at[pl.ds(s * DPT, DPT)])
    plsc.subcore_barrier()

    def _issue_idx(k, b):
        off = k * CH
        pltpu.async_copy(ei_hbm.at[0, pl.ds(off, CH)], rb[b], isem[b])
        pltpu.async_copy(ei_hbm.at[1, pl.ds(off, CH)], cb[b], isem[b])

    _issue_idx(wid, 0)
    _issue_idx(wid + NW, 1)
    _issue_idx(wid + 2 * NW, 2)

    def _group(g, _):
        for b in range(3):
            k = wid + NW * (g * 3 + b)

            @pl.when(k < NCH)
            def _(b=b, k=k):
                pltpu.make_async_copy(ei_hbm.at[0, pl.ds(0, CH)], rb[b],
                                      isem[b]).wait()
                pltpu.make_async_copy(ei_hbm.at[1, pl.ds(0, CH)], cb[b],
                                      isem[b]).wait()

                def _vec(j, _):
                    r = rb[b][pl.ds(j * 16, 16)]
                    cc = cb[b][pl.ds(j * 16, 16)]
                    vb[b][pl.ds(j * 16, 16)] = jnp.where(r != cc, 1.0, 0.0)
                    return 0

                lax.fori_loop(0, CH // 16, _vec, 0)
                pltpu.sync_copy(vb[b], deg_sh.at[rb[b]], add=True)

                @pl.when(k + 3 * NW < NCH)
                def _():
                    _issue_idx(k + 3 * NW, b)

        return 0

    lax.fori_loop(0, _NG, _group, 0)
    plsc.subcore_barrier()
    pltpu.sync_copy(deg_sh.at[pl.ds(s * DPT, DPT)],
                    deg_out.at[c, pl.ds(s * DPT, DPT)])


_deg_kernel = functools.partial(
    pl.kernel,
    out_type=jax.ShapeDtypeStruct((NC, DEG_ROWS), jnp.float32),
    mesh=_mesh,
    scratch_types=(
        [pltpu.VMEM((CH,), jnp.int32)] * 6
        + [pltpu.VMEM((CH,), jnp.float32)] * 3
        + [pltpu.VMEM((DPT,), jnp.float32),
           pltpu.VMEM_SHARED((DEG_ROWS,), jnp.float32)]
        + [pltpu.SemaphoreType.DMA] * 3
    ),
)(_deg_body)


_NB = 3
_NG = (MAXI + _NB - 1) // _NB


def _spmm_body(y_hbm, ei_hbm, acc_out,
               rb0, rb1, rb2, cb0, cb1, cb2,
               mb0, mb1, mb2, acc_sh,
               is0, is1, is2, gs0, gs1, gs2, ss0, ss1, ss2):
    c = lax.axis_index("c")
    s = lax.axis_index("s")
    wid = s * NC + c
    rb = (rb0, rb1, rb2)
    cb = (cb0, cb1, cb2)
    mb = (mb0, mb1, mb2)
    isem = (is0, is1, is2)
    gsem = (gs0, gs1, gs2)
    ssem = (ss0, ss1, ss2)

    def _zz(i, _):
        def _zrow(j, _):
            mb0[i, pl.ds(j * 16, 16)] = jnp.zeros((16,), jnp.float32)
            return 0

        lax.fori_loop(0, DD // 16, _zrow, 0)
        return 0

    lax.fori_loop(0, CH, _zz, 0)

    def _zacc(k, _):
        pltpu.sync_copy(mb0, acc_sh.at[pl.ds(s * RPT + k * CH, CH)])
        return 0

    lax.fori_loop(0, RPT // CH, _zacc, 0)
    pltpu.sync_copy(mb0.at[pl.ds(0, RPT % CH)],
                    acc_sh.at[pl.ds(s * RPT + (RPT // CH) * CH, RPT % CH)])
    plsc.subcore_barrier()

    def _issue_idx(k, b):
        off = k * CH
        pltpu.async_copy(ei_hbm.at[0, pl.ds(off, CH)], rb[b], isem[b])
        pltpu.async_copy(ei_hbm.at[1, pl.ds(off, CH)], cb[b], isem[b])

    def _issue_gather(b):
        pltpu.make_async_copy(ei_hbm.at[0, pl.ds(0, CH)], rb[b],
                              isem[b]).wait()
        pltpu.make_async_copy(ei_hbm.at[1, pl.ds(0, CH)], cb[b],
                              isem[b]).wait()
        pltpu.async_copy(y_hbm.at[rb[b]], mb[b], gsem[b])

    _issue_idx(wid, 0)
    _issue_idx(wid + NW, 1)
    _issue_gather(0)
    _issue_idx(wid + 2 * NW, 2)
    _issue_gather(1)

    def _group(g, _):
        for b in range(_NB):
            i = g * _NB + b
            k = wid + NW * i

            @pl.when(k < NCH)
            def _(b=b, k=k):
                pltpu.make_async_copy(
                    y_hbm.at[pl.ds(0, CH)], mb[b], gsem[b]).wait()

                def _vec(j, _):
                    r = rb[b][pl.ds(j * 16, 16)]
                    cc = cb[b][pl.ds(j * 16, 16)]
                    cb[b][pl.ds(j * 16, 16)] = jnp.where(r != cc, cc, DUMMY)
                    return 0

                lax.fori_loop(0, CH // 16, _vec, 0)
                pltpu.sync_copy(mb[b], acc_sh.at[cb[b]], add=True)

                @pl.when(k + 3 * NW < NCH)
                def _():
                    _issue_idx(k + 3 * NW, b)

                @pl.when(k + 2 * NW < NCH)
                def _(b=b):
                    _issue_gather((b + 2) % _NB)

        return 0

    lax.fori_loop(0, _NG, _group, 0)
    plsc.subcore_barrier()
    pltpu.sync_copy(acc_sh.at[pl.ds(s * RPT, RPT)],
                    acc_out.at[c, pl.ds(s * RPT, RPT)])


_spmm_kernel = functools.partial(
    pl.kernel,
    out_type=jax.ShapeDtypeStruct((NC, ACC_ROWS, DD), jnp.float32),
    mesh=_mesh,
    scratch_types=(
        [pltpu.VMEM((CH,), jnp.int32)] * 6
        + [pltpu.VMEM((CH, DD), jnp.float32)] * 3
        + [pltpu.VMEM_SHARED((ACC_ROWS, DD), jnp.float32)]
        + [pltpu.SemaphoreType.DMA] * 9
    ),
)(_spmm_body)


@jax.jit
def _impl(edge_index, features, preference):
    ei = edge_index.astype(jnp.int32)
    xcat = jnp.concatenate([preference.astype(jnp.float32),
                            features.astype(jnp.float32)], axis=0)
    x = _tc_norm(xcat)
    deg_parts = _deg_kernel(ei)
    dd = deg_parts[:, :NN].reshape(NC, NN, 1)
    y1 = _tc_scale(x, dd)
    acc1 = _spmm_kernel(y1, ei)
    t1, y2 = _tc_combine_mid(acc1, dd, x)
    acc2 = _spmm_kernel(y2, ei)
    total = _tc_combine_fin(acc2, dd, t1)
    return total, preference


def kernel(edge_index, features, preference):
    return _impl(edge_index, features, preference)


# speedup vs baseline: 1.0994x; 1.0157x over previous
import functools

import jax
import jax.numpy as jnp
from jax import lax
from jax.experimental import pallas as pl
from jax.experimental.pallas import tpu as pltpu
from jax.experimental.pallas import tpu_sc as plsc

NN = 10000
DD = 128
EE = 320000
CH = 128
NCH = EE // CH
NC, NS = 2, 16
NW = NC * NS
MAXI = (NCH + NW - 1) // NW
ACC_ROWS = 10112
RPT = ACC_ROWS // NS
DEG_ROWS = 10240
DPT = DEG_ROWS // NS
ZR = 64
DUMMY = NN

_mesh = plsc.VectorSubcoreMesh(
    core_axis_name="c", subcore_axis_name="s", num_cores=NC, num_subcores=NS
)


def _norm_body(x_ref, o_ref):
    v = x_ref[...]
    n = jnp.sqrt(jnp.sum(v * v, axis=1, keepdims=True))
    o_ref[...] = v / jnp.maximum(n, 1e-12)


def _scale_body(x_ref, dd_ref, y_out):
    dis = lax.rsqrt(dd_ref[0] + dd_ref[1])
    y_out[...] = dis * x_ref[...]


def _combine_mid_body(acc_ref, dd_ref, x_ref, t_ref, y_ref):
    dis = lax.rsqrt(dd_ref[0] + dd_ref[1])
    h = dis * (acc_ref[0] + acc_ref[1])
    t_ref[...] = x_ref[...] + h
    y_ref[...] = dis * h


def _combine_fin_body(acc_ref, dd_ref, t_ref, o_ref):
    dis = lax.rsqrt(dd_ref[0] + dd_ref[1])
    o_ref[...] = t_ref[...] + dis * (acc_ref[0] + acc_ref[1])


_RB = 2000


def _row_spec(width):
    return pl.BlockSpec((_RB, width), lambda i: (i, 0))


_dd_spec = pl.BlockSpec((NC, _RB, 1), lambda i: (0, i, 0))
_acc_spec = pl.BlockSpec((NC, _RB, DD), lambda i: (0, i, 0))
_f32 = functools.partial(jax.ShapeDtypeStruct, dtype=jnp.float32)


def _tc_norm(x):
    return pl.pallas_call(
        _norm_body,
        grid=(NN // _RB,),
        in_specs=[_row_spec(DD)],
        out_specs=_row_spec(DD),
        out_shape=_f32((NN, DD)),
    )(x)


def _tc_scale(x, dd):
    return pl.pallas_call(
        _scale_body,
        grid=(NN // _RB,),
        in_specs=[_row_spec(DD), _dd_spec],
        out_specs=_row_spec(DD),
        out_shape=_f32((NN, DD)),
    )(x, dd)


def _tc_combine_mid(acc, dd, x):
    return pl.pallas_call(
        _combine_mid_body,
        grid=(NN // _RB,),
        in_specs=[_acc_spec, _dd_spec, _row_spec(DD)],
        out_specs=(_row_spec(DD), _row_spec(DD)),
        out_shape=(_f32((NN, DD)), _f32((NN, DD))),
    )(acc, dd, x)


def _tc_combine_fin(acc, dd, t):
    return pl.pallas_call(
        _combine_fin_body,
        grid=(NN // _RB,),
        in_specs=[_acc_spec, _dd_spec, _row_spec(DD)],
        out_specs=_row_spec(DD),
        out_shape=_f32((NN, DD)),
    )(acc, dd, t)


def _deg_body(ei_hbm, deg_out,
              rb0, rb1, rb2, cb0, cb1, cb2, vb0, vb1, vb2, zbuf,
              deg_sh, is0, is1, is2):
    c = lax.axis_index("c")
    s = lax.axis_index("s")
    wid = s * NC + c
    rb = (rb0, rb1, rb2)
    cb = (cb0, cb1, cb2)
    vb = (vb0, vb1, vb2)
    isem = (is0, is1, is2)

    def _zz(i, _):
        zbuf[pl.ds(i * 16, 16)] = jnp.zeros((16,), jnp.float32)
        return 0

    lax.fori_loop(0, DPT // 16, _zz, 0)
    pltpu.sync_copy(zbuf, deg_sh.at[pl.ds(s * DPT, DPT)])
    plsc.subcore_barrier()

    def _issue_idx(k, b):
        off = k * CH
        pltpu.async_copy(ei_hbm.at[0, pl.ds(off, CH)], rb[b], isem[b])
        pltpu.async_copy(ei_hbm.at[1, pl.ds(off, CH)], cb[b], isem[b])

    _issue_idx(wid, 0)
    _issue_idx(wid + NW, 1)
    _issue_idx(wid + 2 * NW, 2)

    def _group(g, _):
        for b in range(3):
            k = wid + NW * (g * 3 + b)

            @pl.when(k < NCH)
            def _(b=b, k=k):
                pltpu.make_async_copy(ei_hbm.at[0, pl.ds(0, CH)], rb[b],
                                      isem[b]).wait()
                pltpu.make_async_copy(ei_hbm.at[1, pl.ds(0, CH)], cb[b],
                                      isem[b]).wait()

                def _vec(j, _):
                    r = rb[b][pl.ds(j * 16, 16)]
                    cc = cb[b][pl.ds(j * 16, 16)]
                    vb[b][pl.ds(j * 16, 16)] = jnp.where(r != cc, 1.0, 0.0)
                    return 0

                lax.fori_loop(0, CH // 16, _vec, 0)
                pltpu.sync_copy(vb[b], deg_sh.at[rb[b]], add=True)

                @pl.when(k + 3 * NW < NCH)
                def _():
                    _issue_idx(k + 3 * NW, b)

        return 0

    lax.fori_loop(0, _NG, _group, 0)
    plsc.subcore_barrier()
    pltpu.sync_copy(deg_sh.at[pl.ds(s * DPT, DPT)],
                    deg_out.at[c, pl.ds(s * DPT, DPT)])


_deg_kernel = functools.partial(
    pl.kernel,
    out_type=jax.ShapeDtypeStruct((NC, DEG_ROWS), jnp.float32),
    mesh=_mesh,
    scratch_types=(
        [pltpu.VMEM((CH,), jnp.int32)] * 6
        + [pltpu.VMEM((CH,), jnp.float32)] * 3
        + [pltpu.VMEM((DPT,), jnp.float32),
           pltpu.VMEM_SHARED((DEG_ROWS,), jnp.float32)]
        + [pltpu.SemaphoreType.DMA] * 3
    ),
)(_deg_body)


_NB = 3
_NG = (MAXI + _NB - 1) // _NB


def _spmm_body(y_hbm, ei_hbm, acc_out,
               rb0, rb1, rb2, cb0, cb1, cb2,
               mb0, mb1, mb2, acc_sh,
               is0, is1, is2, gs0, gs1, gs2, ss0, ss1, ss2):
    c = lax.axis_index("c")
    s = lax.axis_index("s")
    wid = s * NC + c
    rb = (rb0, rb1, rb2)
    cb = (cb0, cb1, cb2)
    mb = (mb0, mb1, mb2)
    isem = (is0, is1, is2)
    gsem = (gs0, gs1, gs2)
    ssem = (ss0, ss1, ss2)

    def _zz(i, _):
        def _zrow(j, _):
            mb0[i, pl.ds(j * 16, 16)] = jnp.zeros((16,), jnp.float32)
            return 0

        lax.fori_loop(0, DD // 16, _zrow, 0)
        return 0

    lax.fori_loop(0, CH, _zz, 0)

    def _zacc(k, _):
        pltpu.sync_copy(mb0, acc_sh.at[pl.ds(s * RPT + k * CH, CH)])
        return 0

    lax.fori_loop(0, RPT // CH, _zacc, 0)
    pltpu.sync_copy(mb0.at[pl.ds(0, RPT % CH)],
                    acc_sh.at[pl.ds(s * RPT + (RPT // CH) * CH, RPT % CH)])
    plsc.subcore_barrier()

    def _issue_idx(k, b):
        off = k * CH
        pltpu.async_copy(ei_hbm.at[0, pl.ds(off, CH)], rb[b], isem[b])
        pltpu.async_copy(ei_hbm.at[1, pl.ds(off, CH)], cb[b], isem[b])

    def _issue_gather(b):
        pltpu.make_async_copy(ei_hbm.at[0, pl.ds(0, CH)], rb[b],
                              isem[b]).wait()
        pltpu.make_async_copy(ei_hbm.at[1, pl.ds(0, CH)], cb[b],
                              isem[b]).wait()
        pltpu.async_copy(y_hbm.at[rb[b]], mb[b], gsem[b])

    _issue_idx(wid, 0)
    _issue_idx(wid + NW, 1)
    _issue_gather(0)
    _issue_idx(wid + 2 * NW, 2)
    _issue_gather(1)

    def _group(g, _):
        for b in range(_NB):
            i = g * _NB + b
            k = wid + NW * i

            @pl.when(k < NCH)
            def _(b=b, k=k):
                pltpu.make_async_copy(
                    y_hbm.at[pl.ds(0, CH)], mb[b], gsem[b]).wait()

                def _vec(j, _):
                    r = rb[b][pl.ds(j * 16, 16)]
                    cc = cb[b][pl.ds(j * 16, 16)]
                    cb[b][pl.ds(j * 16, 16)] = jnp.where(r != cc, cc, DUMMY)
                    return 0

                lax.fori_loop(0, CH // 16, _vec, 0)
                pltpu.sync_copy(mb[b], acc_sh.at[cb[b]], add=True)

                @pl.when(k + 3 * NW < NCH)
                def _():
                    _issue_idx(k + 3 * NW, b)

                @pl.when(k + 2 * NW < NCH)
                def _(b=b):
                    _issue_gather((b + 2) % _NB)

        return 0

    lax.fori_loop(0, _NG, _group, 0)
    plsc.subcore_barrier()
    pltpu.sync_copy(acc_sh.at[pl.ds(s * RPT, RPT)],
                    acc_out.at[c, pl.ds(s * RPT, RPT)])


_spmm_kernel = functools.partial(
    pl.kernel,
    out_type=jax.ShapeDtypeStruct((NC, ACC_ROWS, DD), jnp.float32),
    mesh=_mesh,
    scratch_types=(
        [pltpu.VMEM((CH,), jnp.int32)] * 6
        + [pltpu.VMEM((CH, DD), jnp.float32)] * 3
        + [pltpu.VMEM_SHARED((ACC_ROWS, DD), jnp.float32)]
        + [pltpu.SemaphoreType.DMA] * 9
    ),
)(_spmm_body)


@jax.jit
def _impl(edge_index, features, preference):
    ei = edge_index.astype(jnp.int32)
    xcat = jnp.concatenate([preference.astype(jnp.float32),
                            features.astype(jnp.float32)], axis=0)
    x = _tc_norm(xcat)
    deg_parts = _deg_kernel(ei)
    dd = deg_parts[:, :NN].reshape(NC, NN, 1)
    y1 = _tc_scale(x, dd)
    acc1 = _spmm_kernel(y1, ei)
    t1, y2 = _tc_combine_mid(acc1, dd, x)
    acc2 = _spmm_kernel(y2, ei)
    total = _tc_combine_fin(acc2, dd, t1)
    return total, preference


def kernel(edge_index, features, preference):
    return _impl(edge_index, features, preference)


# speedup vs baseline: 1.1036x; 1.0038x over previous
import functools

import jax
import jax.numpy as jnp
from jax import lax
from jax.experimental import pallas as pl
from jax.experimental.pallas import tpu as pltpu
from jax.experimental.pallas import tpu_sc as plsc

NN = 10000
DD = 128
EE = 320000
CH = 128
NCH = EE // CH
NC, NS = 2, 16
NW = NC * NS
MAXI = (NCH + NW - 1) // NW
ACC_ROWS = 10112
RPT = ACC_ROWS // NS
DEG_ROWS = 10240
DPT = DEG_ROWS // NS
ZR = 64
DUMMY = NN

_mesh = plsc.VectorSubcoreMesh(
    core_axis_name="c", subcore_axis_name="s", num_cores=NC, num_subcores=NS
)


def _norm_body(x_ref, o_ref):
    v = x_ref[...]
    n = jnp.sqrt(jnp.sum(v * v, axis=1, keepdims=True))
    o_ref[...] = v / jnp.maximum(n, 1e-12)


def _scale_body(x_ref, dd_ref, y_out):
    dis = lax.rsqrt(dd_ref[0] + dd_ref[1])
    y_out[...] = dis * x_ref[...]


def _combine_mid_body(acc_ref, dd_ref, x_ref, t_ref, y_ref):
    dis = lax.rsqrt(dd_ref[0] + dd_ref[1])
    h = dis * (acc_ref[0] + acc_ref[1])
    t_ref[...] = x_ref[...] + h
    y_ref[...] = dis * h


def _combine_fin_body(acc_ref, dd_ref, t_ref, o_ref):
    dis = lax.rsqrt(dd_ref[0] + dd_ref[1])
    o_ref[...] = t_ref[...] + dis * (acc_ref[0] + acc_ref[1])


_RB = 5000


def _row_spec(width):
    return pl.BlockSpec((_RB, width), lambda i: (i, 0))


_dd_spec = pl.BlockSpec((NC, _RB, 1), lambda i: (0, i, 0))
_acc_spec = pl.BlockSpec((NC, _RB, DD), lambda i: (0, i, 0))
_f32 = functools.partial(jax.ShapeDtypeStruct, dtype=jnp.float32)


def _tc_norm(x):
    return pl.pallas_call(
        _norm_body,
        grid=(NN // _RB,),
        in_specs=[_row_spec(DD)],
        out_specs=_row_spec(DD),
        out_shape=_f32((NN, DD)),
    )(x)


def _tc_scale(x, dd):
    return pl.pallas_call(
        _scale_body,
        grid=(NN // _RB,),
        in_specs=[_row_spec(DD), _dd_spec],
        out_specs=_row_spec(DD),
        out_shape=_f32((NN, DD)),
    )(x, dd)


def _tc_combine_mid(acc, dd, x):
    return pl.pallas_call(
        _combine_mid_body,
        grid=(NN // _RB,),
        in_specs=[_acc_spec, _dd_spec, _row_spec(DD)],
        out_specs=(_row_spec(DD), _row_spec(DD)),
        out_shape=(_f32((NN, DD)), _f32((NN, DD))),
    )(acc, dd, x)


def _tc_combine_fin(acc, dd, t):
    return pl.pallas_call(
        _combine_fin_body,
        grid=(NN // _RB,),
        in_specs=[_acc_spec, _dd_spec, _row_spec(DD)],
        out_specs=_row_spec(DD),
        out_shape=_f32((NN, DD)),
    )(acc, dd, t)


def _deg_body(ei_hbm, deg_out,
              rb0, rb1, rb2, cb0, cb1, cb2, vb0, vb1, vb2, zbuf,
              deg_sh, is0, is1, is2):
    c = lax.axis_index("c")
    s = lax.axis_index("s")
    wid = s * NC + c
    rb = (rb0, rb1, rb2)
    cb = (cb0, cb1, cb2)
    vb = (vb0, vb1, vb2)
    isem = (is0, is1, is2)

    def _zz(i, _):
        zbuf[pl.ds(i * 16, 16)] = jnp.zeros((16,), jnp.float32)
        return 0

    lax.fori_loop(0, DPT // 16, _zz, 0)
    pltpu.sync_copy(zbuf, deg_sh.at[pl.ds(s * DPT, DPT)])
    plsc.subcore_barrier()

    def _issue_idx(k, b):
        off = k * CH
        pltpu.async_copy(ei_hbm.at[0, pl.ds(off, CH)], rb[b], isem[b])
        pltpu.async_copy(ei_hbm.at[1, pl.ds(off, CH)], cb[b], isem[b])

    _issue_idx(wid, 0)
    _issue_idx(wid + NW, 1)
    _issue_idx(wid + 2 * NW, 2)

    def _group(g, _):
        for b in range(3):
            k = wid + NW * (g * 3 + b)

            @pl.when(k < NCH)
            def _(b=b, k=k):
                pltpu.make_async_copy(ei_hbm.at[0, pl.ds(0, CH)], rb[b],
                                      isem[b]).wait()
                pltpu.make_async_copy(ei_hbm.at[1, pl.ds(0, CH)], cb[b],
                                      isem[b]).wait()

                def _vec(j, _):
                    r = rb[b][pl.ds(j * 16, 16)]
                    cc = cb[b][pl.ds(j * 16, 16)]
                    vb[b][pl.ds(j * 16, 16)] = jnp.where(r != cc, 1.0, 0.0)
                    return 0

                lax.fori_loop(0, CH // 16, _vec, 0)
                pltpu.sync_copy(vb[b], deg_sh.at[rb[b]], add=True)

                @pl.when(k + 3 * NW < NCH)
                def _():
                    _issue_idx(k + 3 * NW, b)

        return 0

    lax.fori_loop(0, _NG, _group, 0)
    plsc.subcore_barrier()
    pltpu.sync_copy(deg_sh.at[pl.ds(s * DPT, DPT)],
                    deg_out.at[c, pl.ds(s * DPT, DPT)])


_deg_kernel = functools.partial(
    pl.kernel,
    out_type=jax.ShapeDtypeStruct((NC, DEG_ROWS), jnp.float32),
    mesh=_mesh,
    scratch_types=(
        [pltpu.VMEM((CH,), jnp.int32)] * 6
        + [pltpu.VMEM((CH,), jnp.float32)] * 3
        + [pltpu.VMEM((DPT,), jnp.float32),
           pltpu.VMEM_SHARED((DEG_ROWS,), jnp.float32)]
        + [pltpu.SemaphoreType.DMA] * 3
    ),
)(_deg_body)


_NB = 3
_NG = (MAXI + _NB - 1) // _NB


def _spmm_body(y_hbm, ei_hbm, acc_out,
               rb0, rb1, rb2, cb0, cb1, cb2,
               mb0, mb1, mb2, acc_sh,
               is0, is1, is2, gs0, gs1, gs2, ss0, ss1, ss2):
    c = lax.axis_index("c")
    s = lax.axis_index("s")
    wid = s * NC + c
    rb = (rb0, rb1, rb2)
    cb = (cb0, cb1, cb2)
    mb = (mb0, mb1, mb2)
    isem = (is0, is1, is2)
    gsem = (gs0, gs1, gs2)
    ssem = (ss0, ss1, ss2)

    def _zz(i, _):
        def _zrow(j, _):
            mb0[i, pl.ds(j * 16, 16)] = jnp.zeros((16,), jnp.float32)
            return 0

        lax.fori_loop(0, DD // 16, _zrow, 0)
        return 0

    lax.fori_loop(0, CH, _zz, 0)

    def _zacc(k, _):
        pltpu.sync_copy(mb0, acc_sh.at[pl.ds(s * RPT + k * CH, CH)])
        return 0

    lax.fori_loop(0, RPT // CH, _zacc, 0)
    pltpu.sync_copy(mb0.at[pl.ds(0, RPT % CH)],
                    acc_sh.at[pl.ds(s * RPT + (RPT // CH) * CH, RPT % CH)])
    plsc.subcore_barrier()

    def _issue_idx(k, b):
        off = k * CH
        pltpu.async_copy(ei_hbm.at[0, pl.ds(off, CH)], rb[b], isem[b])
        pltpu.async_copy(ei_hbm.at[1, pl.ds(off, CH)], cb[b], isem[b])

    def _issue_gather(b):
        pltpu.make_async_copy(ei_hbm.at[0, pl.ds(0, CH)], rb[b],
                              isem[b]).wait()
        pltpu.make_async_copy(ei_hbm.at[1, pl.ds(0, CH)], cb[b],
                              isem[b]).wait()
        pltpu.async_copy(y_hbm.at[rb[b]], mb[b], gsem[b])

    _issue_idx(wid, 0)
    _issue_idx(wid + NW, 1)
    _issue_gather(0)
    _issue_idx(wid + 2 * NW, 2)
    _issue_gather(1)

    def _group(g, _):
        for b in range(_NB):
            i = g * _NB + b
            k = wid + NW * i

            @pl.when(k < NCH)
            def _(b=b, k=k):
                pltpu.make_async_copy(
                    y_hbm.at[pl.ds(0, CH)], mb[b], gsem[b]).wait()

                def _vec(j, _):
                    r = rb[b][pl.ds(j * 16, 16)]
                    cc = cb[b][pl.ds(j * 16, 16)]
                    cb[b][pl.ds(j * 16, 16)] = jnp.where(r != cc, cc, DUMMY)
                    return 0

                lax.fori_loop(0, CH // 16, _vec, 0)
                pltpu.sync_copy(mb[b], acc_sh.at[cb[b]], add=True)

                @pl.when(k + 3 * NW < NCH)
                def _():
                    _issue_idx(k + 3 * NW, b)

                @pl.when(k + 2 * NW < NCH)
                def _(b=b):
                    _issue_gather((b + 2) % _NB)

        return 0

    lax.fori_loop(0, _NG, _group, 0)
    plsc.subcore_barrier()
    pltpu.sync_copy(acc_sh.at[pl.ds(s * RPT, RPT)],
                    acc_out.at[c, pl.ds(s * RPT, RPT)])


_spmm_kernel = functools.partial(
    pl.kernel,
    out_type=jax.ShapeDtypeStruct((NC, ACC_ROWS, DD), jnp.float32),
    mesh=_mesh,
    scratch_types=(
        [pltpu.VMEM((CH,), jnp.int32)] * 6
        + [pltpu.VMEM((CH, DD), jnp.float32)] * 3
        + [pltpu.VMEM_SHARED((ACC_ROWS, DD), jnp.float32)]
        + [pltpu.SemaphoreType.DMA] * 9
    ),
)(_spmm_body)


@jax.jit
def _impl(edge_index, features, preference):
    ei = edge_index.astype(jnp.int32)
    xcat = jnp.concatenate([preference.astype(jnp.float32),
                            features.astype(jnp.float32)], axis=0)
    x = _tc_norm(xcat)
    deg_parts = _deg_kernel(ei)
    dd = deg_parts[:, :NN].reshape(NC, NN, 1)
    y1 = _tc_scale(x, dd)
    acc1 = _spmm_kernel(y1, ei)
    t1, y2 = _tc_combine_mid(acc1, dd, x)
    acc2 = _spmm_kernel(y2, ei)
    total = _tc_combine_fin(acc2, dd, t1)
    return total, preference


def kernel(edge_index, features, preference):
    return _impl(edge_index, features, preference)
